# baseline (speedup 1.0000x reference)
import functools

import jax
import jax.numpy as jnp
from jax import lax
from jax.experimental import pallas as pl
from jax.experimental.pallas import tpu as pltpu
from jax.experimental.pallas import tpu_sc as plsc

N = 10000
E = 320000
D = 128
NG = 16

NC = 2
NS = 16
NW = NC * NS
L = 128
NCH = 80
EPS = NCH * L
EP = NW * EPS
ER = EP // L
NP = 10240
RPT = NP // NS
ROWBLK = 2000
NBLK = N // ROWBLK

_F32 = jnp.float32
_HIGH = jax.lax.Precision.HIGHEST


def _mesh():
    return plsc.VectorSubcoreMesh(
        core_axis_name="c", subcore_axis_name="s", num_cores=NC, num_subcores=NS
    )


@functools.partial(
    pl.kernel,
    out_type=jax.ShapeDtypeStruct((NC * NP,), _F32),
    mesh=_mesh(),
    scratch_types=[
        pltpu.VMEM((NCH, L), jnp.int32),
        pltpu.VMEM((NCH, L), _F32),
        pltpu.VMEM((L,), _F32),
        pltpu.VMEM_SHARED((NP,), _F32),
    ],
)
def _sc_deg(dst2_hbm, ew2_hbm, out_hbm, dst_v, ew_v, zero_v, acc_sh):
    c = lax.axis_index("c")
    s = lax.axis_index("s")
    z16 = jnp.zeros((16,), _F32)
    for g in range(L // 16):
        zero_v[pl.ds(16 * g, 16)] = z16
    for i in range(RPT // L):
        pltpu.sync_copy(zero_v, acc_sh.at[pl.ds(s * RPT + i * L, L)])
    plsc.subcore_barrier()

    rowbase = (c * NS + s) * NCH
    pltpu.sync_copy(dst2_hbm.at[pl.ds(rowbase, NCH)], dst_v)
    pltpu.sync_copy(ew2_hbm.at[pl.ds(rowbase, NCH)], ew_v)

    def chunk_body(j, _):
        pltpu.sync_copy(ew_v.at[j], acc_sh.at[dst_v.at[j]], add=True)
        return 0

    lax.fori_loop(0, NCH, chunk_body, 0)
    plsc.subcore_barrier()

    for i in range(RPT // L):
        off = s * RPT + i * L
        pltpu.sync_copy(acc_sh.at[pl.ds(off, L)],
                        out_hbm.at[pl.ds(c * NP + off, L)])


@functools.partial(
    pl.kernel,
    out_type=jax.ShapeDtypeStruct((NC, NP, D), _F32),
    mesh=_mesh(),
    scratch_types=[
        pltpu.VMEM((NCH, L), jnp.int32),
        pltpu.VMEM((L,), jnp.int32),
        pltpu.VMEM((L,), jnp.int32),
        pltpu.VMEM((L,), _F32),
        pltpu.VMEM((L,), _F32),
        pltpu.VMEM((L, D), _F32),
        pltpu.VMEM((L, D), _F32),
        pltpu.VMEM_SHARED((NP, D), _F32),
        pltpu.SemaphoreType.DMA,
        pltpu.SemaphoreType.DMA,
    ],
)
def _sc_spmm(u_hbm, src2_hbm, dst2_hbm, ew2_hbm, out_hbm,
             src_v, dst_a, dst_b, ew_a, ew_b, rows_a, rows_b, acc_sh,
             sem_a, sem_b):
    c = lax.axis_index("c")
    s = lax.axis_index("s")
    z16 = jnp.zeros((16,), _F32)

    def zero_rows(e, _):
        for j in range(D // 16):
            rows_a[e, pl.ds(16 * j, 16)] = z16
        return 0

    lax.fori_loop(0, L, zero_rows, 0)
    for i in range(RPT // L):
        pltpu.sync_copy(rows_a, acc_sh.at[pl.ds(s * RPT + i * L, L)])
    plsc.subcore_barrier()

    rowbase = (c * NS + s) * NCH
    pltpu.sync_copy(src2_hbm.at[pl.ds(rowbase, NCH)], src_v)

    H = L // 2

    def load(j, dst_v, ew_v, rows_v, sem):
        pltpu.async_copy(dst2_hbm.at[rowbase + j], dst_v, sem)
        pltpu.async_copy(ew2_hbm.at[rowbase + j], ew_v, sem)
        pltpu.async_copy(u_hbm.at[src_v.at[j, pl.ds(0, H)]],
                         rows_v.at[pl.ds(0, H)], sem)
        pltpu.async_copy(u_hbm.at[src_v.at[j, pl.ds(H, H)]],
                         rows_v.at[pl.ds(H, H)], sem)

    def wait(dst_v, ew_v, rows_v, sem):
        pltpu.make_async_copy(dst2_hbm.at[0], dst_v, sem).wait()
        pltpu.make_async_copy(ew2_hbm.at[0], ew_v, sem).wait()
        pltpu.make_async_copy(u_hbm.at[src_v.at[0, pl.ds(0, H)]],
                              rows_v.at[pl.ds(0, H)], sem).wait()
        pltpu.make_async_copy(u_hbm.at[src_v.at[0, pl.ds(H, H)]],
                              rows_v.at[pl.ds(H, H)], sem).wait()

    def process(dst_v, ew_v, rows_v):
        def scale_body(g, _):
            wv = ew_v[pl.ds(16 * g, 16)]
            for l in range(16):
                e = 16 * g + l
                w = jnp.full((16,), wv[l], _F32)
                for t in range(D // 16):
                    sl = pl.ds(16 * t, 16)
                    rows_v[e, sl] = rows_v[e, sl] * w
            return 0

        lax.fori_loop(0, L // 16, scale_body, 0)
        pltpu.sync_copy(rows_v, acc_sh.at[dst_v], add=True)

    load(0, dst_a, ew_a, rows_a, sem_a)
    load(1, dst_b, ew_b, rows_b, sem_b)

    def pair_body(i, _):
        k = 2 * i
        wait(dst_a, ew_a, rows_a, sem_a)
        process(dst_a, ew_a, rows_a)
        load(k + 2, dst_a, ew_a, rows_a, sem_a)
        wait(dst_b, ew_b, rows_b, sem_b)
        process(dst_b, ew_b, rows_b)
        load(k + 3, dst_b, ew_b, rows_b, sem_b)
        return 0

    lax.fori_loop(0, (NCH - 2) // 2, pair_body, 0)

    wait(dst_a, ew_a, rows_a, sem_a)
    process(dst_a, ew_a, rows_a)
    wait(dst_b, ew_b, rows_b, sem_b)
    process(dst_b, ew_b, rows_b)

    plsc.subcore_barrier()
    for i in range(RPT // L):
        off = s * RPT + i * L
        pltpu.sync_copy(acc_sh.at[pl.ds(off, L)],
                        out_hbm.at[c, pl.ds(off, L)])


def _tc_prep_body(x_ref, w1_ref, degt_ref, u1_ref, dinv_ref):
    deg = jnp.sum(degt_ref[...], axis=1, keepdims=True) + 1.0
    dinv = jax.lax.rsqrt(deg)
    dinv_ref[...] = dinv
    z = jnp.dot(x_ref[...], w1_ref[...],
                preferred_element_type=_F32, precision=_HIGH)
    u1_ref[...] = z * dinv


def _tc_prep(x, W1, degt):
    return pl.pallas_call(
        _tc_prep_body,
        out_shape=[
            jax.ShapeDtypeStruct((N, D), _F32),
            jax.ShapeDtypeStruct((N, 1), _F32),
        ],
    )(x, W1, degt)


def _tc_mid_body(p0_ref, p1_ref, u1_ref, dinv_ref, b1_ref, w2_ref, u2_ref):
    dinv = dinv_ref[...]
    h1 = jnp.maximum(
        dinv * (p0_ref[...] + p1_ref[...] + u1_ref[...]) + b1_ref[...], 0.0)
    z = jnp.dot(h1, w2_ref[...], preferred_element_type=_F32, precision=_HIGH)
    u2_ref[...] = z * dinv


def _tc_mid(p0, p1, u1, dinv, b1, W2):
    row = pl.BlockSpec((ROWBLK, D), lambda i: (i, 0))
    return pl.pallas_call(
        _tc_mid_body,
        grid=(NBLK,),
        in_specs=[
            row, row, row,
            pl.BlockSpec((ROWBLK, 1), lambda i: (i, 0)),
            pl.BlockSpec((1, D), lambda i: (0, 0)),
            pl.BlockSpec((D, D), lambda i: (0, 0)),
        ],
        out_specs=row,
        out_shape=jax.ShapeDtypeStruct((N, D), _F32),
    )(p0, p1, u1, dinv, b1, W2)


def _tc_out_body(p0_ref, p1_ref, u2_ref, dinv_ref, b2_ref, batch_ref,
                 h_ref, pooled_ref, counts_ref):
    i = pl.program_id(0)

    @pl.when(i == 0)
    def _():
        pooled_ref[...] = jnp.zeros((NG, D), _F32)
        counts_ref[...] = jnp.zeros((NG, 1), _F32)

    h = jnp.maximum(
        dinv_ref[...] * (p0_ref[...] + p1_ref[...] + u2_ref[...])
        + b2_ref[...], 0.0)
    h_ref[...] = h

    gids = lax.broadcasted_iota(jnp.int32, (ROWBLK, NG), 1)
    mask = (batch_ref[...] == gids).astype(_F32)
    pooled_ref[...] += lax.dot_general(
        mask, h, (((0,), (0,)), ((), ())),
        preferred_element_type=_F32, precision=_HIGH)
    counts_ref[...] += lax.dot_general(
        mask, jnp.ones((ROWBLK, 1), _F32), (((0,), (0,)), ((), ())),
        preferred_element_type=_F32, precision=_HIGH)

    @pl.when(i == NBLK - 1)
    def _():
        pooled_ref[...] = pooled_ref[...] / jnp.maximum(counts_ref[...], 1.0)


def _tc_out(p0, p1, u2, dinv, b2, batch2d):
    row = pl.BlockSpec((ROWBLK, D), lambda i: (i, 0))
    return pl.pallas_call(
        _tc_out_body,
        grid=(NBLK,),
        in_specs=[
            row, row, row,
            pl.BlockSpec((ROWBLK, 1), lambda i: (i, 0)),
            pl.BlockSpec((1, D), lambda i: (0, 0)),
            pl.BlockSpec((ROWBLK, 1), lambda i: (i, 0)),
        ],
        out_specs=[
            row,
            pl.BlockSpec((NG, D), lambda i: (0, 0)),
        ],
        out_shape=[
            jax.ShapeDtypeStruct((N, D), _F32),
            jax.ShapeDtypeStruct((NG, D), _F32),
        ],
        scratch_shapes=[pltpu.VMEM((NG, 1), _F32)],
    )(p0, p1, u2, dinv, b2, batch2d)


def kernel(x, edge_index, edge_weight, batch, W1, b1, W2, b2):
    src = edge_index[0]
    dst = edge_index[1]

    padi = jnp.zeros((EP - E,), jnp.int32)
    padf = jnp.zeros((EP - E,), _F32)
    src2 = jnp.concatenate([src, padi]).reshape(ER, L)
    dst2 = jnp.concatenate([dst, padi]).reshape(ER, L)
    ew2 = jnp.concatenate([edge_weight, padf]).reshape(ER, L)

    degp = _sc_deg(dst2, ew2).reshape(NC, NP)
    degt = jnp.transpose(degp[:, :N])
    u1, dinv = _tc_prep(x, W1, degt)

    s1p = _sc_spmm(u1, src2, dst2, ew2)
    u2 = _tc_mid(s1p[0, :N], s1p[1, :N], u1, dinv,
                 b1.reshape(1, D), W2)

    s2p = _sc_spmm(u2, src2, dst2, ew2)
    h2, pooled = _tc_out(s2p[0, :N], s2p[1, :N], u2, dinv,
                         b2.reshape(1, D), batch.reshape(N, 1))
    return (h2, pooled)


# speedup vs baseline: 1.0011x; 1.0011x over previous
import functools

import jax
import jax.numpy as jnp
from jax import lax
from jax.experimental import pallas as pl
from jax.experimental.pallas import tpu as pltpu
from jax.experimental.pallas import tpu_sc as plsc

N = 10000
E = 320000
D = 128
NG = 16

NC = 2
NS = 16
NW = NC * NS
L = 128
NCH = 80
EPS = NCH * L
EP = NW * EPS
ER = EP // L
NP = 10240
RPT = NP // NS
ROWBLK = 2000
NBLK = N // ROWBLK

_F32 = jnp.float32
_HIGH = jax.lax.Precision.HIGHEST


def _mesh():
    return plsc.VectorSubcoreMesh(
        core_axis_name="c", subcore_axis_name="s", num_cores=NC, num_subcores=NS
    )


@functools.partial(
    pl.kernel,
    out_type=jax.ShapeDtypeStruct((NC * NP,), _F32),
    mesh=_mesh(),
    scratch_types=[
        pltpu.VMEM((NCH, L), jnp.int32),
        pltpu.VMEM((NCH, L), _F32),
        pltpu.VMEM((L,), _F32),
        pltpu.VMEM_SHARED((NP,), _F32),
    ],
)
def _sc_deg(dst2_hbm, ew2_hbm, out_hbm, dst_v, ew_v, zero_v, acc_sh):
    c = lax.axis_index("c")
    s = lax.axis_index("s")
    z16 = jnp.zeros((16,), _F32)
    for g in range(L // 16):
        zero_v[pl.ds(16 * g, 16)] = z16
    for i in range(RPT // L):
        pltpu.sync_copy(zero_v, acc_sh.at[pl.ds(s * RPT + i * L, L)])
    plsc.subcore_barrier()

    rowbase = (c * NS + s) * NCH
    pltpu.sync_copy(dst2_hbm.at[pl.ds(rowbase, NCH)], dst_v)
    pltpu.sync_copy(ew2_hbm.at[pl.ds(rowbase, NCH)], ew_v)

    def chunk_body(j, _):
        pltpu.sync_copy(ew_v.at[j], acc_sh.at[dst_v.at[j]], add=True)
        return 0

    lax.fori_loop(0, NCH, chunk_body, 0)
    plsc.subcore_barrier()

    for i in range(RPT // L):
        off = s * RPT + i * L
        pltpu.sync_copy(acc_sh.at[pl.ds(off, L)],
                        out_hbm.at[pl.ds(c * NP + off, L)])


@functools.partial(
    pl.kernel,
    out_type=jax.ShapeDtypeStruct((NC, NP, D), _F32),
    mesh=_mesh(),
    scratch_types=[
        pltpu.VMEM((NCH, L), jnp.int32),
        pltpu.VMEM((L,), jnp.int32),
        pltpu.VMEM((L,), jnp.int32),
        pltpu.VMEM((L,), _F32),
        pltpu.VMEM((L,), _F32),
        pltpu.VMEM((L, D), _F32),
        pltpu.VMEM((L, D), _F32),
        pltpu.VMEM_SHARED((NP, D), _F32),
        pltpu.SemaphoreType.DMA,
        pltpu.SemaphoreType.DMA,
    ],
)
def _sc_spmm(u_hbm, src2_hbm, dst2_hbm, ew2_hbm, out_hbm,
             src_v, dst_a, dst_b, ew_a, ew_b, rows_a, rows_b, acc_sh,
             sem_a, sem_b):
    c = lax.axis_index("c")
    s = lax.axis_index("s")
    z16 = jnp.zeros((16,), _F32)

    def zero_rows(e, _):
        for j in range(D // 16):
            rows_a[e, pl.ds(16 * j, 16)] = z16
        return 0

    lax.fori_loop(0, L, zero_rows, 0)
    for i in range(RPT // L):
        pltpu.sync_copy(rows_a, acc_sh.at[pl.ds(s * RPT + i * L, L)])
    plsc.subcore_barrier()

    rowbase = (c * NS + s) * NCH
    pltpu.sync_copy(src2_hbm.at[pl.ds(rowbase, NCH)], src_v)

    def load(j, dst_v, ew_v, rows_v, sem):
        pltpu.async_copy(dst2_hbm.at[rowbase + j], dst_v, sem)
        pltpu.async_copy(ew2_hbm.at[rowbase + j], ew_v, sem)
        pltpu.async_copy(u_hbm.at[src_v.at[j]], rows_v, sem)

    def wait(dst_v, ew_v, rows_v, sem):
        pltpu.make_async_copy(dst2_hbm.at[0], dst_v, sem).wait()
        pltpu.make_async_copy(ew2_hbm.at[0], ew_v, sem).wait()
        pltpu.make_async_copy(u_hbm.at[src_v.at[0]], rows_v, sem).wait()

    def process(dst_v, ew_v, rows_v):
        def scale_body(g, _):
            wv = ew_v[pl.ds(16 * g, 16)]
            for l in range(16):
                e = 16 * g + l
                w = jnp.full((16,), wv[l], _F32)
                for t in range(D // 16):
                    sl = pl.ds(16 * t, 16)
                    rows_v[e, sl] = rows_v[e, sl] * w
            return 0

        lax.fori_loop(0, L // 16, scale_body, 0)
        pltpu.sync_copy(rows_v, acc_sh.at[dst_v], add=True)

    load(0, dst_a, ew_a, rows_a, sem_a)
    load(1, dst_b, ew_b, rows_b, sem_b)

    def pair_body(i, _):
        k = 2 * i
        wait(dst_a, ew_a, rows_a, sem_a)
        process(dst_a, ew_a, rows_a)
        load(k + 2, dst_a, ew_a, rows_a, sem_a)
        wait(dst_b, ew_b, rows_b, sem_b)
        process(dst_b, ew_b, rows_b)
        load(k + 3, dst_b, ew_b, rows_b, sem_b)
        return 0

    lax.fori_loop(0, (NCH - 2) // 2, pair_body, 0)

    wait(dst_a, ew_a, rows_a, sem_a)
    process(dst_a, ew_a, rows_a)
    wait(dst_b, ew_b, rows_b, sem_b)
    process(dst_b, ew_b, rows_b)

    plsc.subcore_barrier()
    for i in range(RPT // L):
        off = s * RPT + i * L
        pltpu.sync_copy(acc_sh.at[pl.ds(off, L)],
                        out_hbm.at[c, pl.ds(off, L)])


def _tc_prep_body(x_ref, w1_ref, degt_ref, u1_ref, dinv_ref):
    deg = jnp.sum(degt_ref[...], axis=1, keepdims=True) + 1.0
    dinv = jax.lax.rsqrt(deg)
    dinv_ref[...] = dinv
    z = jnp.dot(x_ref[...], w1_ref[...],
                preferred_element_type=_F32, precision=_HIGH)
    u1_ref[...] = z * dinv


def _tc_prep(x, W1, degt):
    return pl.pallas_call(
        _tc_prep_body,
        out_shape=[
            jax.ShapeDtypeStruct((N, D), _F32),
            jax.ShapeDtypeStruct((N, 1), _F32),
        ],
    )(x, W1, degt)


def _tc_mid_body(p0_ref, p1_ref, u1_ref, dinv_ref, b1_ref, w2_ref, u2_ref):
    dinv = dinv_ref[...]
    h1 = jnp.maximum(
        dinv * (p0_ref[...] + p1_ref[...] + u1_ref[...]) + b1_ref[...], 0.0)
    z = jnp.dot(h1, w2_ref[...], preferred_element_type=_F32, precision=_HIGH)
    u2_ref[...] = z * dinv


def _tc_mid(p0, p1, u1, dinv, b1, W2):
    row = pl.BlockSpec((ROWBLK, D), lambda i: (i, 0))
    return pl.pallas_call(
        _tc_mid_body,
        grid=(NBLK,),
        in_specs=[
            row, row, row,
            pl.BlockSpec((ROWBLK, 1), lambda i: (i, 0)),
            pl.BlockSpec((1, D), lambda i: (0, 0)),
            pl.BlockSpec((D, D), lambda i: (0, 0)),
        ],
        out_specs=row,
        out_shape=jax.ShapeDtypeStruct((N, D), _F32),
    )(p0, p1, u1, dinv, b1, W2)


def _tc_out_body(p0_ref, p1_ref, u2_ref, dinv_ref, b2_ref, batch_ref,
                 h_ref, pooled_ref, counts_ref):
    i = pl.program_id(0)

    @pl.when(i == 0)
    def _():
        pooled_ref[...] = jnp.zeros((NG, D), _F32)
        counts_ref[...] = jnp.zeros((NG, 1), _F32)

    h = jnp.maximum(
        dinv_ref[...] * (p0_ref[...] + p1_ref[...] + u2_ref[...])
        + b2_ref[...], 0.0)
    h_ref[...] = h

    gids = lax.broadcasted_iota(jnp.int32, (ROWBLK, NG), 1)
    mask = (batch_ref[...] == gids).astype(_F32)
    pooled_ref[...] += lax.dot_general(
        mask, h, (((0,), (0,)), ((), ())),
        preferred_element_type=_F32, precision=_HIGH)
    counts_ref[...] += lax.dot_general(
        mask, jnp.ones((ROWBLK, 1), _F32), (((0,), (0,)), ((), ())),
        preferred_element_type=_F32, precision=_HIGH)

    @pl.when(i == NBLK - 1)
    def _():
        pooled_ref[...] = pooled_ref[...] / jnp.maximum(counts_ref[...], 1.0)


def _tc_out(p0, p1, u2, dinv, b2, batch2d):
    row = pl.BlockSpec((ROWBLK, D), lambda i: (i, 0))
    return pl.pallas_call(
        _tc_out_body,
        grid=(NBLK,),
        in_specs=[
            row, row, row,
            pl.BlockSpec((ROWBLK, 1), lambda i: (i, 0)),
            pl.BlockSpec((1, D), lambda i: (0, 0)),
            pl.BlockSpec((ROWBLK, 1), lambda i: (i, 0)),
        ],
        out_specs=[
            row,
            pl.BlockSpec((NG, D), lambda i: (0, 0)),
        ],
        out_shape=[
            jax.ShapeDtypeStruct((N, D), _F32),
            jax.ShapeDtypeStruct((NG, D), _F32),
        ],
        scratch_shapes=[pltpu.VMEM((NG, 1), _F32)],
    )(p0, p1, u2, dinv, b2, batch2d)


def kernel(x, edge_index, edge_weight, batch, W1, b1, W2, b2):
    src = edge_index[0]
    dst = edge_index[1]

    padi = jnp.zeros((EP - E,), jnp.int32)
    padf = jnp.zeros((EP - E,), _F32)
    src2 = jnp.concatenate([src, padi]).reshape(ER, L)
    dst2 = jnp.concatenate([dst, padi]).reshape(ER, L)
    ew2 = jnp.concatenate([edge_weight, padf]).reshape(ER, L)

    degp = _sc_deg(dst2, ew2).reshape(NC, NP)
    degt = jnp.transpose(degp[:, :N])
    u1, dinv = _tc_prep(x, W1, degt)

    s1p = _sc_spmm(u1, src2, dst2, ew2)
    u2 = _tc_mid(s1p[0, :N], s1p[1, :N], u1, dinv,
                 b1.reshape(1, D), W2)

    s2p = _sc_spmm(u2, src2, dst2, ew2)
    h2, pooled = _tc_out(s2p[0, :N], s2p[1, :N], u2, dinv,
                         b2.reshape(1, D), batch.reshape(N, 1))
    return (h2, pooled)


# speedup vs baseline: 1.0022x; 1.0012x over previous
import functools

import jax
import jax.numpy as jnp
from jax import lax
from jax.experimental import pallas as pl
from jax.experimental.pallas import tpu as pltpu
from jax.experimental.pallas import tpu_sc as plsc

N = 10000
E = 320000
D = 128
NG = 16

NC = 2
NS = 16
NW = NC * NS
L = 128
NCH = 80
EPS = NCH * L
EP = NW * EPS
ER = EP // L
NP = 10240
RPT = NP // NS
ROWBLK = 2000
NBLK = N // ROWBLK

_F32 = jnp.float32
_HIGH = jax.lax.Precision.HIGHEST


def _mesh():
    return plsc.VectorSubcoreMesh(
        core_axis_name="c", subcore_axis_name="s", num_cores=NC, num_subcores=NS
    )


@functools.partial(
    pl.kernel,
    out_type=jax.ShapeDtypeStruct((NC * NP,), _F32),
    mesh=_mesh(),
    scratch_types=[
        pltpu.VMEM((NCH, L), jnp.int32),
        pltpu.VMEM((NCH, L), _F32),
        pltpu.VMEM((L,), _F32),
        pltpu.VMEM_SHARED((NP,), _F32),
    ],
)
def _sc_deg(dst2_hbm, ew2_hbm, out_hbm, dst_v, ew_v, zero_v, acc_sh):
    c = lax.axis_index("c")
    s = lax.axis_index("s")
    z16 = jnp.zeros((16,), _F32)
    for g in range(L // 16):
        zero_v[pl.ds(16 * g, 16)] = z16
    for i in range(RPT // L):
        pltpu.sync_copy(zero_v, acc_sh.at[pl.ds(s * RPT + i * L, L)])
    plsc.subcore_barrier()

    rowbase = (c * NS + s) * NCH
    pltpu.sync_copy(dst2_hbm.at[pl.ds(rowbase, NCH)], dst_v)
    pltpu.sync_copy(ew2_hbm.at[pl.ds(rowbase, NCH)], ew_v)

    def chunk_body(j, _):
        pltpu.sync_copy(ew_v.at[j], acc_sh.at[dst_v.at[j]], add=True)
        return 0

    lax.fori_loop(0, NCH, chunk_body, 0)
    plsc.subcore_barrier()

    for i in range(RPT // L):
        off = s * RPT + i * L
        pltpu.sync_copy(acc_sh.at[pl.ds(off, L)],
                        out_hbm.at[pl.ds(c * NP + off, L)])


@functools.partial(
    pl.kernel,
    out_type=jax.ShapeDtypeStruct((NC, NP, D), _F32),
    mesh=_mesh(),
    scratch_types=[
        pltpu.VMEM((NCH, L), jnp.int32),
        pltpu.VMEM((L,), jnp.int32),
        pltpu.VMEM((L,), jnp.int32),
        pltpu.VMEM((L,), _F32),
        pltpu.VMEM((L,), _F32),
        pltpu.VMEM((L, D), _F32),
        pltpu.VMEM((L, D), _F32),
        pltpu.VMEM_SHARED((NP, D), _F32),
        pltpu.SemaphoreType.DMA,
        pltpu.SemaphoreType.DMA,
    ],
)
def _sc_spmm(u_hbm, src2_hbm, dst2_hbm, ew2_hbm, out_hbm,
             src_v, dst_a, dst_b, ew_a, ew_b, rows_a, rows_b, acc_sh,
             sem_a, sem_b):
    c = lax.axis_index("c")
    s = lax.axis_index("s")
    z16 = jnp.zeros((16,), _F32)

    def zero_rows(e, _):
        for j in range(D // 16):
            rows_a[e, pl.ds(16 * j, 16)] = z16
        return 0

    lax.fori_loop(0, L, zero_rows, 0)
    for i in range(RPT // L):
        pltpu.sync_copy(rows_a, acc_sh.at[pl.ds(s * RPT + i * L, L)])
    plsc.subcore_barrier()

    rowbase = (c * NS + s) * NCH
    pltpu.sync_copy(src2_hbm.at[pl.ds(rowbase, NCH)], src_v)

    def load(j, dst_v, ew_v, rows_v, sem):
        pltpu.async_copy(dst2_hbm.at[rowbase + j], dst_v, sem)
        pltpu.async_copy(ew2_hbm.at[rowbase + j], ew_v, sem)
        pltpu.async_copy(u_hbm.at[src_v.at[j]], rows_v, sem)

    def wait(dst_v, ew_v, rows_v, sem):
        pltpu.make_async_copy(dst2_hbm.at[0], dst_v, sem).wait()
        pltpu.make_async_copy(ew2_hbm.at[0], ew_v, sem).wait()
        pltpu.make_async_copy(u_hbm.at[src_v.at[0]], rows_v, sem).wait()

    def process(dst_v, ew_v, rows_v):
        def scale_body(g, _):
            wv = ew_v[pl.ds(16 * g, 16)]
            for l in range(16):
                e = 16 * g + l
                w = jnp.full((16,), wv[l], _F32)
                for t in range(D // 16):
                    sl = pl.ds(16 * t, 16)
                    rows_v[e, sl] = rows_v[e, sl] * w
            return 0

        lax.fori_loop(0, L // 16, scale_body, 0)
        pltpu.sync_copy(rows_v, acc_sh.at[dst_v], add=True)

    load(0, dst_a, ew_a, rows_a, sem_a)
    load(1, dst_b, ew_b, rows_b, sem_b)

    def pair_body(i, _):
        k = 2 * i
        wait(dst_a, ew_a, rows_a, sem_a)
        process(dst_a, ew_a, rows_a)
        load(k + 2, dst_a, ew_a, rows_a, sem_a)
        wait(dst_b, ew_b, rows_b, sem_b)
        process(dst_b, ew_b, rows_b)
        load(k + 3, dst_b, ew_b, rows_b, sem_b)
        return 0

    lax.fori_loop(0, (NCH - 2) // 2, pair_body, 0)

    wait(dst_a, ew_a, rows_a, sem_a)
    process(dst_a, ew_a, rows_a)
    wait(dst_b, ew_b, rows_b, sem_b)
    process(dst_b, ew_b, rows_b)

    plsc.subcore_barrier()
    for i in range(RPT // L):
        off = s * RPT + i * L
        pltpu.sync_copy(acc_sh.at[pl.ds(off, L)],
                        out_hbm.at[c, pl.ds(off, L)])


def _tc_prep_body(x_ref, w1_ref, degt_ref, u1_ref, dinv_ref):
    deg = jnp.sum(degt_ref[...], axis=1, keepdims=True) + 1.0
    dinv = jax.lax.rsqrt(deg)
    dinv_ref[...] = dinv
    z = jnp.dot(x_ref[...], w1_ref[...],
                preferred_element_type=_F32, precision=_HIGH)
    u1_ref[...] = z * dinv


def _tc_prep(x, W1, degt):
    return pl.pallas_call(
        _tc_prep_body,
        out_shape=[
            jax.ShapeDtypeStruct((N, D), _F32),
            jax.ShapeDtypeStruct((N, 1), _F32),
        ],
    )(x, W1, degt)


def _tc_mid_body(p0_ref, p1_ref, u1_ref, dinv_ref, b1_ref, w2_ref, u2_ref):
    dinv = dinv_ref[...]
    h1 = jnp.maximum(
        dinv * (p0_ref[...] + p1_ref[...] + u1_ref[...]) + b1_ref[...], 0.0)
    z = jnp.dot(h1, w2_ref[...], preferred_element_type=_F32, precision=_HIGH)
    u2_ref[...] = z * dinv


def _tc_mid(p0, p1, u1, dinv, b1, W2):
    row = pl.BlockSpec((ROWBLK, D), lambda i: (i, 0))
    return pl.pallas_call(
        _tc_mid_body,
        grid=(NBLK,),
        in_specs=[
            row, row, row,
            pl.BlockSpec((ROWBLK, 1), lambda i: (i, 0)),
            pl.BlockSpec((1, D), lambda i: (0, 0)),
            pl.BlockSpec((D, D), lambda i: (0, 0)),
        ],
        out_specs=row,
        out_shape=jax.ShapeDtypeStruct((N, D), _F32),
    )(p0, p1, u1, dinv, b1, W2)


def _tc_out_body(p0_ref, p1_ref, u2_ref, dinv_ref, b2_ref, batch_ref,
                 h_ref, pooled_ref, counts_ref):
    i = pl.program_id(0)

    @pl.when(i == 0)
    def _():
        pooled_ref[...] = jnp.zeros((NG, D), _F32)
        counts_ref[...] = jnp.zeros((NG, 1), _F32)

    h = jnp.maximum(
        dinv_ref[...] * (p0_ref[...] + p1_ref[...] + u2_ref[...])
        + b2_ref[...], 0.0)
    h_ref[...] = h

    gids = lax.broadcasted_iota(jnp.int32, (ROWBLK, NG), 1)
    mask = (batch_ref[...] == gids).astype(_F32)
    pooled_ref[...] += lax.dot_general(
        mask, h, (((0,), (0,)), ((), ())),
        preferred_element_type=_F32, precision=_HIGH)
    counts_ref[...] += lax.dot_general(
        mask, jnp.ones((ROWBLK, 1), _F32), (((0,), (0,)), ((), ())),
        preferred_element_type=_F32, precision=_HIGH)

    @pl.when(i == NBLK - 1)
    def _():
        pooled_ref[...] = pooled_ref[...] / jnp.maximum(counts_ref[...], 1.0)


def _tc_out(p0, p1, u2, dinv, b2, batch2d):
    row = pl.BlockSpec((ROWBLK, D), lambda i: (i, 0))
    return pl.pallas_call(
        _tc_out_body,
        grid=(NBLK,),
        in_specs=[
            row, row, row,
            pl.BlockSpec((ROWBLK, 1), lambda i: (i, 0)),
            pl.BlockSpec((1, D), lambda i: (0, 0)),
            pl.BlockSpec((ROWBLK, 1), lambda i: (i, 0)),
        ],
        out_specs=[
            row,
            pl.BlockSpec((NG, D), lambda i: (0, 0)),
        ],
        out_shape=[
            jax.ShapeDtypeStruct((N, D), _F32),
            jax.ShapeDtypeStruct((NG, D), _F32),
        ],
        scratch_shapes=[pltpu.VMEM((NG, 1), _F32)],
    )(p0, p1, u2, dinv, b2, batch2d)


def kernel(x, edge_index, edge_weight, batch, W1, b1, W2, b2):
    src = edge_index[0]
    dst = edge_index[1]

    padi = jnp.zeros((EP - E,), jnp.int32)
    padd = N + (jnp.arange(EP - E, dtype=jnp.int32) % (NP - N))
    padf = jnp.zeros((EP - E,), _F32)
    src2 = jnp.concatenate([src, padi]).reshape(ER, L)
    dst2 = jnp.concatenate([dst, padd]).reshape(ER, L)
    ew2 = jnp.concatenate([edge_weight, padf]).reshape(ER, L)

    degp = _sc_deg(dst2, ew2).reshape(NC, NP)
    degt = jnp.transpose(degp[:, :N])
    u1, dinv = _tc_prep(x, W1, degt)

    s1p = _sc_spmm(u1, src2, dst2, ew2)
    u2 = _tc_mid(s1p[0, :N], s1p[1, :N], u1, dinv,
                 b1.reshape(1, D), W2)

    s2p = _sc_spmm(u2, src2, dst2, ew2)
    h2, pooled = _tc_out(s2p[0, :N], s2p[1, :N], u2, dinv,
                         b2.reshape(1, D), batch.reshape(N, 1))
    return (h2, pooled)
